# Initial kernel scaffold; baseline (speedup 1.0000x reference)
import jax
import jax.numpy as jnp
from jax.experimental import pallas as pl


def kernel(task_id, table, W1, b1, gamma, beta, W2, b2):
    raise NotImplementedError("write your pallas kernel here")


# speedup vs baseline: 1.9957x; 1.9957x over previous
import functools

import jax
import jax.numpy as jnp
from jax import lax
from jax.experimental import pallas as pl
from jax.experimental.pallas import tpu as pltpu
from jax.experimental.pallas import tpu_sc as plsc

NUM_TASKS = 100000
EMBED_DIM = 128
BATCH = 16384

_NC = 2
_NS = 16
_NW = _NC * _NS
_B_PER_W = BATCH // _NW


@functools.lru_cache(maxsize=1)
def _make_sc_gather():
    mesh = plsc.VectorSubcoreMesh(core_axis_name="c", subcore_axis_name="s")

    @functools.partial(
        pl.kernel,
        mesh=mesh,
        out_type=jax.ShapeDtypeStruct((BATCH, EMBED_DIM), jnp.float32),
        scratch_types=[
            pltpu.VMEM((_B_PER_W,), jnp.int32),
            pltpu.VMEM((_B_PER_W, EMBED_DIM), jnp.float32),
            pltpu.SemaphoreType.DMA,
        ],
    )
    def sc_gather(table_hbm, idx_hbm, out_hbm, idx_v, rows_v, sem):
        wid = lax.axis_index("s") * _NC + lax.axis_index("c")
        base = wid * _B_PER_W
        pltpu.sync_copy(idx_hbm.at[pl.ds(base, _B_PER_W)], idx_v)
        pltpu.async_copy(table_hbm.at[idx_v], rows_v, sem).wait()
        pltpu.sync_copy(rows_v, out_hbm.at[pl.ds(base, _B_PER_W)])

    return sc_gather


_ROW_BLK = 1024


def _mlp_body(e_ref, w1_ref, b1_ref, gamma_ref, beta_ref, w2_ref, b2_ref,
              out_ref):
    e = e_ref[...]
    h = lax.dot_general(e, w1_ref[...], (((1,), (1,)), ((), ())),
                        preferred_element_type=jnp.float32)
    h = h + b1_ref[...]
    mu = jnp.mean(h, axis=-1, keepdims=True)
    d = h - mu
    var = jnp.mean(d * d, axis=-1, keepdims=True)
    h = d * lax.rsqrt(var + 1e-5) * gamma_ref[...] + beta_ref[...]
    h = 0.5 * h * (1.0 + lax.erf(h * 0.7071067811865476))
    out = lax.dot_general(h, w2_ref[...], (((1,), (1,)), ((), ())),
                          preferred_element_type=jnp.float32)
    out_ref[...] = out + b2_ref[...]


def _mlp(e, W1, b1, gamma, beta, W2, b2):
    grid = (BATCH // _ROW_BLK,)
    full = pl.BlockSpec((EMBED_DIM, EMBED_DIM), lambda i: (0, 0))
    vec = pl.BlockSpec((1, EMBED_DIM), lambda i: (0, 0))
    return pl.pallas_call(
        _mlp_body,
        grid=grid,
        in_specs=[
            pl.BlockSpec((_ROW_BLK, EMBED_DIM), lambda i: (i, 0)),
            full, vec, vec, vec, full, vec,
        ],
        out_specs=pl.BlockSpec((_ROW_BLK, EMBED_DIM), lambda i: (i, 0)),
        out_shape=jax.ShapeDtypeStruct((BATCH, EMBED_DIM), jnp.float32),
    )(e, W1, b1.reshape(1, EMBED_DIM), gamma.reshape(1, EMBED_DIM),
      beta.reshape(1, EMBED_DIM), W2, b2.reshape(1, EMBED_DIM))


@jax.jit
def kernel(task_id, table, W1, b1, gamma, beta, W2, b2):
    e = _make_sc_gather()(table, task_id.astype(jnp.int32))
    return _mlp(e, W1, b1, gamma, beta, W2, b2)


# speedup vs baseline: 2.0389x; 1.0217x over previous
import functools

import jax
import jax.numpy as jnp
from jax import lax
from jax.experimental import pallas as pl
from jax.experimental.pallas import tpu as pltpu
from jax.experimental.pallas import tpu_sc as plsc

NUM_TASKS = 100000
EMBED_DIM = 128
BATCH = 16384

_NC = 2
_NS = 16
_NW = _NC * _NS
_B_PER_W = BATCH // _NW


@functools.lru_cache(maxsize=1)
def _make_sc_gather():
    mesh = plsc.VectorSubcoreMesh(core_axis_name="c", subcore_axis_name="s")

    @functools.partial(
        pl.kernel,
        mesh=mesh,
        out_type=jax.ShapeDtypeStruct((BATCH, EMBED_DIM), jnp.float32),
        scratch_types=[
            pltpu.VMEM((_B_PER_W,), jnp.int32),
            pltpu.VMEM((_B_PER_W, EMBED_DIM), jnp.float32),
            pltpu.SemaphoreType.DMA,
        ],
    )
    def sc_gather(table_hbm, idx_hbm, out_hbm, idx_v, rows_v, sem):
        wid = lax.axis_index("s") * _NC + lax.axis_index("c")
        base = wid * _B_PER_W
        pltpu.sync_copy(idx_hbm.at[pl.ds(base, _B_PER_W)], idx_v)
        pltpu.async_copy(table_hbm.at[idx_v], rows_v, sem).wait()
        pltpu.sync_copy(rows_v, out_hbm.at[pl.ds(base, _B_PER_W)])

    return sc_gather


_ROW_BLK = 1024


def _mlp_body(e_ref, a_ref, b1c_ref, gamma_ref, beta_ref, w2t_ref, b2_ref,
              out_ref):
    e = e_ref[...]
    hc = lax.dot_general(e, a_ref[...], (((1,), (0,)), ((), ())),
                         preferred_element_type=jnp.float32)
    hc = hc + b1c_ref[...]
    var = jnp.mean(hc * hc, axis=-1, keepdims=True)
    h = hc * lax.rsqrt(var + 1e-5) * gamma_ref[...] + beta_ref[...]
    h = 0.5 * h * (1.0 + lax.erf(h * 0.7071067811865476))
    out = lax.dot_general(h, w2t_ref[...], (((1,), (0,)), ((), ())),
                          preferred_element_type=jnp.float32)
    out_ref[...] = out + b2_ref[...]


def _mlp(e, W1, b1, gamma, beta, W2, b2):
    A = W1.T - jnp.mean(W1, axis=0)[:, None]
    b1c = b1 - jnp.mean(b1)
    grid = (BATCH // _ROW_BLK,)
    full = pl.BlockSpec((EMBED_DIM, EMBED_DIM), lambda i: (0, 0))
    vec = pl.BlockSpec((1, EMBED_DIM), lambda i: (0, 0))
    return pl.pallas_call(
        _mlp_body,
        grid=grid,
        in_specs=[
            pl.BlockSpec((_ROW_BLK, EMBED_DIM), lambda i: (i, 0)),
            full, vec, vec, vec, full, vec,
        ],
        out_specs=pl.BlockSpec((_ROW_BLK, EMBED_DIM), lambda i: (i, 0)),
        out_shape=jax.ShapeDtypeStruct((BATCH, EMBED_DIM), jnp.float32),
    )(e, A, b1c.reshape(1, EMBED_DIM), gamma.reshape(1, EMBED_DIM),
      beta.reshape(1, EMBED_DIM), W2.T, b2.reshape(1, EMBED_DIM))


@jax.jit
def kernel(task_id, table, W1, b1, gamma, beta, W2, b2):
    e = _make_sc_gather()(table, task_id.astype(jnp.int32))
    return _mlp(e, W1, b1, gamma, beta, W2, b2)


# speedup vs baseline: 2.2873x; 1.1218x over previous
import functools

import jax
import jax.numpy as jnp
from jax import lax
from jax.experimental import pallas as pl
from jax.experimental.pallas import tpu as pltpu
from jax.experimental.pallas import tpu_sc as plsc

NUM_TASKS = 100000
EMBED_DIM = 128
BATCH = 16384

_NC = 2
_NS = 16
_NW = _NC * _NS
_B_PER_W = BATCH // _NW


@functools.lru_cache(maxsize=1)
def _make_sc_gather():
    mesh = plsc.VectorSubcoreMesh(core_axis_name="c", subcore_axis_name="s")

    @functools.partial(
        pl.kernel,
        mesh=mesh,
        out_type=jax.ShapeDtypeStruct((BATCH, EMBED_DIM), jnp.float32),
        scratch_types=[
            pltpu.VMEM((_B_PER_W,), jnp.int32),
            pltpu.VMEM((_B_PER_W, EMBED_DIM), jnp.float32),
            pltpu.SemaphoreType.DMA,
        ],
    )
    def sc_gather(table_hbm, idx_hbm, out_hbm, idx_v, rows_v, sem):
        wid = lax.axis_index("s") * _NC + lax.axis_index("c")
        base = wid * _B_PER_W
        pltpu.sync_copy(idx_hbm.at[pl.ds(base, _B_PER_W)], idx_v)
        pltpu.async_copy(table_hbm.at[idx_v], rows_v, sem).wait()
        pltpu.sync_copy(rows_v, out_hbm.at[pl.ds(base, _B_PER_W)])

    return sc_gather


_ROW_BLK = 2048


def _mlp_body(e_ref, a_ref, b1c_ref, gamma_ref, beta_ref, w2t_ref, b2_ref,
              out_ref):
    e = e_ref[...]
    hc = lax.dot_general(e, a_ref[...], (((1,), (0,)), ((), ())),
                         preferred_element_type=jnp.float32)
    hc = hc + b1c_ref[...]
    avg = jnp.full((EMBED_DIM, EMBED_DIM), 1.0 / EMBED_DIM, dtype=jnp.float32)
    var = lax.dot_general(hc * hc, avg, (((1,), (0,)), ((), ())),
                          preferred_element_type=jnp.float32)
    h = hc * lax.rsqrt(var + 1e-5) * gamma_ref[...] + beta_ref[...]
    h = 0.5 * h * (1.0 + lax.erf(h * 0.7071067811865476))
    out = lax.dot_general(h, w2t_ref[...], (((1,), (0,)), ((), ())),
                          preferred_element_type=jnp.float32)
    out_ref[...] = out + b2_ref[...]


def _mlp(e, W1, b1, gamma, beta, W2, b2):
    A = W1.T - jnp.mean(W1, axis=0)[:, None]
    b1c = b1 - jnp.mean(b1)
    grid = (BATCH // _ROW_BLK,)
    full = pl.BlockSpec((EMBED_DIM, EMBED_DIM), lambda i: (0, 0))
    vec = pl.BlockSpec((1, EMBED_DIM), lambda i: (0, 0))
    return pl.pallas_call(
        _mlp_body,
        grid=grid,
        in_specs=[
            pl.BlockSpec((_ROW_BLK, EMBED_DIM), lambda i: (i, 0)),
            full, vec, vec, vec, full, vec,
        ],
        out_specs=pl.BlockSpec((_ROW_BLK, EMBED_DIM), lambda i: (i, 0)),
        out_shape=jax.ShapeDtypeStruct((BATCH, EMBED_DIM), jnp.float32),
    )(e, A, b1c.reshape(1, EMBED_DIM), gamma.reshape(1, EMBED_DIM),
      beta.reshape(1, EMBED_DIM), W2.T, b2.reshape(1, EMBED_DIM))


@jax.jit
def kernel(task_id, table, W1, b1, gamma, beta, W2, b2):
    e = _make_sc_gather()(table, task_id.astype(jnp.int32))
    return _mlp(e, W1, b1, gamma, beta, W2, b2)
